# baseline (speedup 1.0000x reference)
import functools

import jax
import jax.numpy as jnp
from jax import lax
from jax.experimental import pallas as pl
from jax.experimental.pallas import tpu as pltpu
from jax.experimental.pallas import tpu_sc as plsc

VOCAB = 100000
DIM = 128
BATCH = 16384

_info = plsc.get_sparse_core_info()
_NC = _info.num_cores
_NS = _info.num_subcores
_L = _info.num_lanes
_NW = _NC * _NS
_BPW = BATCH // _NW
_RPC = BATCH // _NC
_PB = 2048
_NPW = _RPC // _PB


def _tec_body(idx_hbm, table_hbm, prompt_hbm, out_hbm, idx_v, rows_v,
              spmem_p, gsem):
    wid = lax.axis_index("s") * _NC + lax.axis_index("c")
    base = wid * _BPW
    pltpu.sync_copy(idx_hbm.at[pl.ds(base, _BPW)], idx_v)
    pltpu.async_copy(table_hbm.at[idx_v], rows_v, gsem).wait()
    pltpu.sync_copy(rows_v, out_hbm.at[pl.ds(base, _BPW), pl.ds(DIM, DIM)])


def _scs_body(idx_hbm, table_hbm, prompt_hbm, out_hbm, idx_v, rows_v,
              spmem_p, gsem):
    cid = lax.axis_index("c")
    pltpu.sync_copy(prompt_hbm, spmem_p.at[pl.ds(0, 1)])
    n = 1
    while n < _PB:
        pltpu.sync_copy(spmem_p.at[pl.ds(0, n)], spmem_p.at[pl.ds(n, n)])
        n *= 2
    for k in range(_NPW):
        pltpu.sync_copy(
            spmem_p,
            out_hbm.at[pl.ds(cid * _RPC + k * _PB, _PB), pl.ds(0, DIM)])


@jax.jit
def _run(token_ids, table, prompt):
    vmesh = plsc.VectorSubcoreMesh(core_axis_name="c", subcore_axis_name="s")
    smesh = plsc.ScalarSubcoreMesh(axis_name="c")
    f = pl.kernel(
        body=[_tec_body, _scs_body],
        mesh=[vmesh, smesh],
        out_type=jax.ShapeDtypeStruct((BATCH, 2 * DIM), jnp.float32),
        scratch_types=[
            (pltpu.MemorySpace.VMEM @ vmesh)((_BPW,), jnp.int32),
            (pltpu.MemorySpace.VMEM @ vmesh)((_BPW, DIM), jnp.float32),
            pltpu.MemorySpace.VMEM_SHARED((_PB, DIM), jnp.float32),
            pltpu.SemaphoreType.DMA @ vmesh,
        ],
    )
    return f(token_ids, table, prompt)


def kernel(token_ids, table, prompt):
    return _run(token_ids.astype(jnp.int32), table, prompt)


# speedup vs baseline: 3.2562x; 3.2562x over previous
import functools

import jax
import jax.numpy as jnp
from jax import lax
from jax.experimental import pallas as pl
from jax.experimental.pallas import tpu as pltpu
from jax.experimental.pallas import tpu_sc as plsc

VOCAB = 100000
DIM = 128
BATCH = 16384

_info = plsc.get_sparse_core_info()
_NC = _info.num_cores
_NS = _info.num_subcores
_L = _info.num_lanes
_NW = _NC * _NS
_BPW = BATCH // _NW
_PCH = _BPW // 2


def _body(idx_hbm, table_hbm, prompt_hbm, out_hbm, idx_v, rows_v, prompt_v,
          gsem, wsem):
    wid = lax.axis_index("c") * _NS + lax.axis_index("s")
    base = wid * _BPW

    pltpu.sync_copy(idx_hbm.at[pl.ds(base, _BPW)], idx_v)
    gather = pltpu.async_copy(table_hbm.at[idx_v], rows_v, gsem)

    pltpu.sync_copy(prompt_hbm, prompt_v.at[pl.ds(0, 1)])
    pvecs = [prompt_v[0, pl.ds(j * _L, _L)] for j in range(DIM // _L)]

    def fill_row(i, carry):
        for j in range(DIM // _L):
            prompt_v[i, pl.ds(j * _L, _L)] = pvecs[j]
        return carry

    lax.fori_loop(1, _PCH, fill_row, 0)

    w0 = pltpu.async_copy(
        prompt_v, out_hbm.at[pl.ds(base, _PCH), pl.ds(0, DIM)], wsem)
    w1 = pltpu.async_copy(
        prompt_v, out_hbm.at[pl.ds(base + _PCH, _PCH), pl.ds(0, DIM)], wsem)

    gather.wait()
    w2 = pltpu.async_copy(
        rows_v, out_hbm.at[pl.ds(base, _BPW), pl.ds(DIM, DIM)], wsem)
    w0.wait()
    w1.wait()
    w2.wait()


@jax.jit
def _run(token_ids, table, prompt):
    mesh = plsc.VectorSubcoreMesh(core_axis_name="c", subcore_axis_name="s")
    f = functools.partial(
        pl.kernel,
        mesh=mesh,
        out_type=jax.ShapeDtypeStruct((BATCH, 2 * DIM), jnp.float32),
        scratch_types=[
            pltpu.VMEM((_BPW,), jnp.int32),
            pltpu.VMEM((_BPW, DIM), jnp.float32),
            pltpu.VMEM((_PCH, DIM), jnp.float32),
            pltpu.SemaphoreType.DMA,
            pltpu.SemaphoreType.DMA,
        ],
    )(_body)
    return f(token_ids, table, prompt)


def kernel(token_ids, table, prompt):
    return _run(token_ids.astype(jnp.int32), table, prompt)


# speedup vs baseline: 3.2649x; 1.0027x over previous
import functools

import jax
import jax.numpy as jnp
from jax import lax
from jax.experimental import pallas as pl
from jax.experimental.pallas import tpu as pltpu
from jax.experimental.pallas import tpu_sc as plsc

VOCAB = 100000
DIM = 128
BATCH = 16384

_info = plsc.get_sparse_core_info()
_NC = _info.num_cores
_NS = _info.num_subcores
_L = _info.num_lanes
_NW = _NC * _NS
_BPW = BATCH // _NW
_PCH = _BPW // 2


def _body(idx_hbm, table_hbm, prompt_hbm, out_hbm, idx_v, rows_v, prompt_v,
          gsem, wsem):
    wid = lax.axis_index("c") * _NS + lax.axis_index("s")
    base = wid * _BPW

    pltpu.sync_copy(idx_hbm.at[pl.ds(base, _BPW)], idx_v)
    gather = pltpu.async_copy(table_hbm.at[idx_v], rows_v, gsem)

    pltpu.sync_copy(prompt_hbm, prompt_v.at[pl.ds(0, 1)])
    pvecs = [prompt_v[0, pl.ds(j * _L, _L)] for j in range(DIM // _L)]

    def fill_row(i, carry):
        for j in range(DIM // _L):
            prompt_v[i, pl.ds(j * _L, _L)] = pvecs[j]
        return carry

    lax.fori_loop(1, _PCH, fill_row, 0)

    w0 = pltpu.async_copy(
        prompt_v, out_hbm.at[pl.ds(base, _PCH), pl.ds(0, DIM)], wsem)
    w1 = pltpu.async_copy(
        prompt_v, out_hbm.at[pl.ds(base + _PCH, _PCH), pl.ds(0, DIM)], wsem)

    gather.wait()
    w2 = pltpu.async_copy(
        rows_v, out_hbm.at[pl.ds(base, _BPW), pl.ds(DIM, DIM)], wsem)
    w0.wait()
    w1.wait()
    w2.wait()


@jax.jit
def _run(token_ids, table, prompt):
    mesh = plsc.VectorSubcoreMesh(core_axis_name="c", subcore_axis_name="s")
    f = functools.partial(
        pl.kernel,
        mesh=mesh,
        out_type=jax.ShapeDtypeStruct((BATCH, 2 * DIM), jnp.float32),
        scratch_types=[
            pltpu.VMEM((_BPW,), jnp.int32),
            pltpu.VMEM((_BPW, DIM), jnp.float32),
            pltpu.VMEM((_PCH, DIM), jnp.float32),
            pltpu.SemaphoreType.DMA,
            pltpu.SemaphoreType.DMA,
        ],
        compiler_params=pltpu.CompilerParams(use_tc_tiling_on_sc=True),
    )(_body)
    return f(token_ids, table, prompt)


def kernel(token_ids, table, prompt):
    return _run(token_ids.astype(jnp.int32), table, prompt)


# speedup vs baseline: 3.2886x; 1.0073x over previous
import functools

import jax
import jax.numpy as jnp
from jax import lax
from jax.experimental import pallas as pl
from jax.experimental.pallas import tpu as pltpu
from jax.experimental.pallas import tpu_sc as plsc

VOCAB = 100000
DIM = 128
BATCH = 16384

_info = plsc.get_sparse_core_info()
_NC = _info.num_cores
_NS = _info.num_subcores
_L = _info.num_lanes
_NW = _NC * _NS
_BPW = BATCH // _NW
_PCH = _BPW // 2


def _body(idx_hbm, table_hbm, prompt_hbm, out_hbm, idx_v, rows_v, prompt_v,
          gsem, wsem):
    wid = lax.axis_index("c") * _NS + lax.axis_index("s")
    base = wid * _BPW

    pltpu.sync_copy(idx_hbm.at[pl.ds(base, _BPW)], idx_v)
    gather = pltpu.async_copy(table_hbm.at[idx_v], rows_v, gsem)

    pltpu.sync_copy(prompt_hbm, prompt_v.at[pl.ds(0, 1)])
    pvecs = [prompt_v[0, pl.ds(j * _L, _L)] for j in range(DIM // _L)]

    def fill_row(i, carry):
        for j in range(DIM // _L):
            prompt_v[i, pl.ds(j * _L, _L)] = pvecs[j]
        return carry

    lax.fori_loop(1, _PCH, fill_row, 0)

    w0 = pltpu.async_copy(
        prompt_v, out_hbm.at[pl.ds(base, _PCH), pl.ds(0, DIM)], wsem)
    w1 = pltpu.async_copy(
        prompt_v, out_hbm.at[pl.ds(base + _PCH, _PCH), pl.ds(0, DIM)], wsem)

    gather.wait()
    w2 = pltpu.async_copy(
        rows_v, out_hbm.at[pl.ds(base, _BPW), pl.ds(DIM, DIM)], wsem)
    w0.wait()
    w1.wait()
    w2.wait()


@jax.jit
def _run(token_ids, table, prompt):
    mesh = plsc.VectorSubcoreMesh(core_axis_name="c", subcore_axis_name="s")
    f = functools.partial(
        pl.kernel,
        mesh=mesh,
        out_type=jax.ShapeDtypeStruct((BATCH, 2 * DIM), jnp.float32),
        scratch_types=[
            pltpu.VMEM((_BPW,), jnp.int32),
            pltpu.VMEM((_BPW, DIM), jnp.float32),
            pltpu.VMEM((_PCH, DIM), jnp.float32),
            pltpu.SemaphoreType.DMA,
            pltpu.SemaphoreType.DMA,
        ],
    )(_body)
    return f(token_ids, table, prompt)


def kernel(token_ids, table, prompt):
    return _run(token_ids.astype(jnp.int32), table, prompt)
